# baseline (speedup 1.0000x reference)
import functools

import jax
import jax.numpy as jnp
from jax import lax
from jax.experimental import pallas as pl
from jax.experimental.pallas import tpu as pltpu
from jax.experimental.pallas import tpu_sc as plsc

B = 16384
D = 32
LANES = 128
RPL = LANES // D
NC = 2
NS = 16
NW = NC * NS
BPW = B // NW
HB = BPW // 2

_sc_mesh = plsc.VectorSubcoreMesh(core_axis_name="c", subcore_axis_name="s")


@functools.partial(
    pl.kernel,
    mesh=_sc_mesh,
    out_type=[
        jax.ShapeDtypeStruct((B, LANES), jnp.float32),
        jax.ShapeDtypeStruct((B, LANES), jnp.float32),
    ],
    scratch_types=[
        pltpu.VMEM((BPW,), jnp.int32),
        pltpu.VMEM((BPW,), jnp.int32),
        pltpu.VMEM((HB,), jnp.int32),
        pltpu.VMEM((HB,), jnp.int32),
        pltpu.VMEM((HB, LANES), jnp.float32),
        pltpu.VMEM((HB, LANES), jnp.float32),
        pltpu.SemaphoreType.DMA,
    ],
)
def _sc_gather(user_hbm, item_hbm, ut4_hbm, it4_hbm, ue_hbm, ie_hbm,
               uidx_v, iidx_v, uq_v, iq_v, urows_v, irows_v, sem):
    wid = lax.axis_index("s") * NC + lax.axis_index("c")
    base = wid * BPW
    pltpu.sync_copy(user_hbm.at[pl.ds(base, BPW)], uidx_v)
    pltpu.sync_copy(item_hbm.at[pl.ds(base, BPW)], iidx_v)
    for h in range(2):
        off = h * HB
        for c in range(HB // 16):
            s = pl.ds(c * 16, 16)
            uq_v[s] = lax.shift_right_logical(uidx_v[pl.ds(off + c * 16, 16)], 2)
            iq_v[s] = lax.shift_right_logical(iidx_v[pl.ds(off + c * 16, 16)], 2)
        cu = pltpu.async_copy(ut4_hbm.at[uq_v], urows_v, sem)
        ci = pltpu.async_copy(it4_hbm.at[iq_v], irows_v, sem)
        cu.wait()
        ci.wait()
        pltpu.sync_copy(urows_v, ue_hbm.at[pl.ds(base + off, HB)])
        pltpu.sync_copy(irows_v, ie_hbm.at[pl.ds(base + off, HB)])


TB = 2048


def _subrow(big, sel):
    out = big[:, 0:D] * sel[:, 0:1]
    for k in range(1, RPL):
        out = out + big[:, k * D:(k + 1) * D] * sel[:, k:k + 1]
    return out


def _tc_dense_body(ue_ref, ie_ref, usel_ref, isel_ref, wut_ref, wit_ref,
                   w1t_ref, b1_ref, w2t_ref, b2_ref, w3t_ref, bias_ref,
                   out_ref):
    ue = _subrow(ue_ref[...], usel_ref[...])
    ie = _subrow(ie_ref[...], isel_ref[...])
    inter = ue * ie
    fm = (jnp.dot(ue, wut_ref[...], preferred_element_type=jnp.float32)
          + jnp.dot(ie, wit_ref[...], preferred_element_type=jnp.float32))
    h = jnp.maximum(
        jnp.dot(inter, w1t_ref[...], preferred_element_type=jnp.float32)
        + b1_ref[...], 0.0)
    h = jnp.maximum(
        jnp.dot(h, w2t_ref[...], preferred_element_type=jnp.float32)
        + b2_ref[...], 0.0)
    deep = jnp.dot(h, w3t_ref[...], preferred_element_type=jnp.float32)
    logit = fm[:, 0] + deep[:, 0] + bias_ref[0]
    out_ref[...] = 1.0 / (1.0 + jnp.exp(-logit))


def _tc_dense(ue4, ie4, usel, isel, wut, wit, w1t, b1, w2t, b2, w3t, bias):
    grid = (B // TB,)
    return pl.pallas_call(
        _tc_dense_body,
        grid=grid,
        in_specs=[
            pl.BlockSpec((TB, LANES), lambda i: (i, 0)),
            pl.BlockSpec((TB, LANES), lambda i: (i, 0)),
            pl.BlockSpec((TB, RPL), lambda i: (i, 0)),
            pl.BlockSpec((TB, RPL), lambda i: (i, 0)),
            pl.BlockSpec(wut.shape, lambda i: (0, 0)),
            pl.BlockSpec(wit.shape, lambda i: (0, 0)),
            pl.BlockSpec(w1t.shape, lambda i: (0, 0)),
            pl.BlockSpec(b1.shape, lambda i: (0,)),
            pl.BlockSpec(w2t.shape, lambda i: (0, 0)),
            pl.BlockSpec(b2.shape, lambda i: (0,)),
            pl.BlockSpec(w3t.shape, lambda i: (0, 0)),
            pl.BlockSpec(bias.shape, lambda i: (0,)),
        ],
        out_specs=pl.BlockSpec((TB,), lambda i: (i,)),
        out_shape=jax.ShapeDtypeStruct((B,), jnp.float32),
    )(ue4, ie4, usel, isel, wut, wit, w1t, b1, w2t, b2, w3t, bias)


def kernel(user, item, user_table, item_table, fm_W, fm_b, W1, b1, W2, b2, W3, b3):
    user = user.astype(jnp.int32)
    item = item.astype(jnp.int32)
    ut4 = user_table.reshape(-1, LANES)
    it4 = item_table.reshape(-1, LANES)
    ue4, ie4 = _sc_gather(user, item, ut4, it4)
    usel = jax.nn.one_hot(user % RPL, RPL, dtype=jnp.float32)
    isel = jax.nn.one_hot(item % RPL, RPL, dtype=jnp.float32)
    wut = fm_W[:, :D].T
    wit = fm_W[:, D:].T
    bias = (fm_b + b3).reshape((1,))
    return _tc_dense(ue4, ie4, usel, isel, wut, wit, W1.T, b1, W2.T, b2,
                     W3.T, bias)
